# baseline (speedup 1.0000x reference)
import functools

import jax
import jax.numpy as jnp
from jax import lax
from jax.experimental import pallas as pl
from jax.experimental.pallas import tpu as pltpu
from jax.experimental.pallas import tpu_sc as plsc

_N = 100000
_D = 64
_B = 1024
_BN = 2048


def _sc_gather_products(src_idx, rel_idx, Wh, Wt, Wf, Wi):
    info = plsc.get_sparse_core_info()
    nw = info.num_cores * info.num_subcores
    bpw = _B // nw
    mesh = plsc.VectorSubcoreMesh(core_axis_name="c", subcore_axis_name="s")

    @functools.partial(
        pl.kernel,
        out_type=[
            jax.ShapeDtypeStruct((_B, _D), jnp.float32),
            jax.ShapeDtypeStruct((_B, _D), jnp.float32),
        ],
        mesh=mesh,
        compiler_params=pltpu.CompilerParams(use_tc_tiling_on_sc=False),
        scratch_types=[
            pltpu.VMEM((bpw,), jnp.int32),
            pltpu.VMEM((bpw,), jnp.int32),
            pltpu.VMEM((bpw, _D), jnp.float32),
            pltpu.VMEM((bpw, _D), jnp.float32),
            pltpu.VMEM((bpw, _D), jnp.float32),
            pltpu.VMEM((bpw, _D), jnp.float32),
            pltpu.SemaphoreType.DMA,
        ],
    )
    def gather_kernel(src_hbm, rel_hbm, wh_hbm, wt_hbm, wf_hbm, wi_hbm,
                      a_hbm, c_hbm, idx_s, idx_r, hv, fv, iv, tv, sem):
        wid = lax.axis_index("s") * info.num_cores + lax.axis_index("c")
        base = wid * bpw
        pltpu.sync_copy(src_hbm.at[pl.ds(base, bpw)], idx_s)
        pltpu.sync_copy(rel_hbm.at[pl.ds(base, bpw)], idx_r)
        cp1 = pltpu.async_copy(wh_hbm.at[idx_s], hv, sem)
        cp2 = pltpu.async_copy(wf_hbm.at[idx_r], fv, sem)
        cp3 = pltpu.async_copy(wi_hbm.at[idx_r], iv, sem)
        cp4 = pltpu.async_copy(wt_hbm.at[idx_s], tv, sem)
        cp1.wait()
        cp2.wait()
        cp3.wait()
        cp4.wait()
        for i in range(bpw):
            for j in range(_D // 16):
                sl = pl.ds(j * 16, 16)
                hv[i, sl] = hv[i, sl] * fv[i, sl]
                tv[i, sl] = tv[i, sl] * iv[i, sl]
        pltpu.sync_copy(hv, a_hbm.at[pl.ds(base, bpw)])
        pltpu.sync_copy(tv, c_hbm.at[pl.ds(base, bpw)])

    return gather_kernel(src_idx, rel_idx, Wh, Wt, Wf, Wi)


def _block_scores(a_ref, c_ref, wtt_ref, wht_ref):
    dn = (((1,), (0,)), ((), ()))
    return (lax.dot_general(a_ref[...], wtt_ref[...], dn,
                            preferred_element_type=jnp.float32)
            + lax.dot_general(c_ref[...], wht_ref[...], dn,
                              preferred_element_type=jnp.float32)) * 0.5


def _merged_body(a_ref, c_ref, wt_ref, wh_ref, s_ref, att_ref, m_s, l_s):
    swp = pl.program_id(0)
    j = pl.program_id(1)
    s = _block_scores(a_ref, c_ref, wt_ref, wh_ref)

    @pl.when((swp == 0) & (j == 0))
    def _():
        m_s[...] = jnp.full_like(m_s, -jnp.inf)
        l_s[...] = jnp.zeros_like(l_s)

    @pl.when(swp == 0)
    def _():
        s_ref[...] = s
        col = lax.broadcasted_iota(jnp.int32, s.shape, 1) + j * _BN
        valid = col < _N
        m_old = m_s[:, 0:1]
        bmax = jnp.max(jnp.where(valid, s, -jnp.inf), axis=1, keepdims=True)
        m_new = jnp.maximum(m_old, bmax)
        p = jnp.where(valid, jnp.exp(s - m_new), 0.0)
        l_new = l_s[:, 0:1] * jnp.exp(m_old - m_new) + jnp.sum(p, axis=1,
                                                               keepdims=True)
        m_s[...] = jnp.broadcast_to(m_new, m_s.shape)
        l_s[...] = jnp.broadcast_to(l_new, l_s.shape)

    @pl.when(swp == 1)
    def _():
        @pl.when(j == 0)
        def _():
            s_ref[...] = s

        m = m_s[:, 0:1]
        rl = 1.0 / l_s[:, 0:1]
        att_ref[...] = jnp.exp(s - m) * rl


def _cols_body(b_ref, n_ref):
    g = pl.program_id(0)
    j = pl.program_id(1)
    r = lax.broadcasted_iota(jnp.int32, b_ref.shape, 0)
    lane = lax.broadcasted_iota(jnp.int32, b_ref.shape, 1)
    q = (j * 5000 + r) * 128 + lane
    x = q >> 5
    b_loc = (x * 21475) >> 26
    n_ref[...] = (q - b_loc * _N).astype(jnp.float32)
    b_ref[...] = (b_loc + 32 * g).astype(jnp.float32)


def _nb():
    return (_N + _BN - 1) // _BN


def kernel(src_idx, rel_idx, Wh, Wt, Wf, Wi):
    a, c = _sc_gather_products(src_idx, rel_idx, Wh, Wt, Wf, Wi)
    wt_t, wh_t = Wt.T, Wh.T
    nb = _nb()

    scores, att = pl.pallas_call(
        _merged_body,
        grid=(2, nb),
        in_specs=[
            pl.BlockSpec((_B, _D), lambda s, j: (0, 0)),
            pl.BlockSpec((_B, _D), lambda s, j: (0, 0)),
            pl.BlockSpec((_D, _BN), lambda s, j: (0, j)),
            pl.BlockSpec((_D, _BN), lambda s, j: (0, j)),
        ],
        out_specs=[
            pl.BlockSpec((_B, _BN), lambda s, j: (0, j * (1 - s))),
            pl.BlockSpec((_B, _BN), lambda s, j: (0, j * s)),
        ],
        out_shape=[
            jax.ShapeDtypeStruct((_B, _N), jnp.float32),
            jax.ShapeDtypeStruct((_B, _N), jnp.float32),
        ],
        scratch_shapes=[
            pltpu.VMEM((_B, 128), jnp.float32),
            pltpu.VMEM((_B, 128), jnp.float32),
        ],
    )(a, c, wt_t, wh_t)

    ncols = _B * _N // 128
    bcol2, ncol2 = pl.pallas_call(
        _cols_body,
        grid=(_B // 32, 5),
        out_specs=[
            pl.BlockSpec((5000, 128), lambda g, j: (g * 5 + j, 0)),
            pl.BlockSpec((5000, 128), lambda g, j: (g * 5 + j, 0)),
        ],
        out_shape=[
            jax.ShapeDtypeStruct((ncols, 128), jnp.float32),
            jax.ShapeDtypeStruct((ncols, 128), jnp.float32),
        ],
    )()
    cur = jnp.stack([bcol2.reshape(-1), ncol2.reshape(-1)], axis=1)

    return scores, att.reshape(-1), cur


# speedup vs baseline: 1.0310x; 1.0310x over previous
import functools

import jax
import jax.numpy as jnp
from jax import lax
from jax.experimental import pallas as pl
from jax.experimental.pallas import tpu as pltpu
from jax.experimental.pallas import tpu_sc as plsc

_N = 100000
_D = 64
_B = 1024
_BN = 4096


def _sc_gather_products(src_idx, rel_idx, Wh, Wt, Wf, Wi):
    info = plsc.get_sparse_core_info()
    nw = info.num_cores * info.num_subcores
    bpw = _B // nw
    mesh = plsc.VectorSubcoreMesh(core_axis_name="c", subcore_axis_name="s")

    @functools.partial(
        pl.kernel,
        out_type=[
            jax.ShapeDtypeStruct((_B, _D), jnp.float32),
            jax.ShapeDtypeStruct((_B, _D), jnp.float32),
        ],
        mesh=mesh,
        compiler_params=pltpu.CompilerParams(use_tc_tiling_on_sc=False),
        scratch_types=[
            pltpu.VMEM((bpw,), jnp.int32),
            pltpu.VMEM((bpw,), jnp.int32),
            pltpu.VMEM((bpw, _D), jnp.float32),
            pltpu.VMEM((bpw, _D), jnp.float32),
            pltpu.VMEM((bpw, _D), jnp.float32),
            pltpu.VMEM((bpw, _D), jnp.float32),
            pltpu.SemaphoreType.DMA,
        ],
    )
    def gather_kernel(src_hbm, rel_hbm, wh_hbm, wt_hbm, wf_hbm, wi_hbm,
                      a_hbm, c_hbm, idx_s, idx_r, hv, fv, iv, tv, sem):
        wid = lax.axis_index("s") * info.num_cores + lax.axis_index("c")
        base = wid * bpw
        pltpu.sync_copy(src_hbm.at[pl.ds(base, bpw)], idx_s)
        pltpu.sync_copy(rel_hbm.at[pl.ds(base, bpw)], idx_r)
        cp1 = pltpu.async_copy(wh_hbm.at[idx_s], hv, sem)
        cp2 = pltpu.async_copy(wf_hbm.at[idx_r], fv, sem)
        cp3 = pltpu.async_copy(wi_hbm.at[idx_r], iv, sem)
        cp4 = pltpu.async_copy(wt_hbm.at[idx_s], tv, sem)
        cp1.wait()
        cp2.wait()
        cp3.wait()
        cp4.wait()
        for i in range(bpw):
            for j in range(_D // 16):
                sl = pl.ds(j * 16, 16)
                hv[i, sl] = hv[i, sl] * fv[i, sl]
                tv[i, sl] = tv[i, sl] * iv[i, sl]
        pltpu.sync_copy(hv, a_hbm.at[pl.ds(base, bpw)])
        pltpu.sync_copy(tv, c_hbm.at[pl.ds(base, bpw)])

    return gather_kernel(src_idx, rel_idx, Wh, Wt, Wf, Wi)


def _block_scores(a_ref, c_ref, wtt_ref, wht_ref):
    dn = (((1,), (0,)), ((), ()))
    return (lax.dot_general(a_ref[...], wtt_ref[...], dn,
                            preferred_element_type=jnp.float32)
            + lax.dot_general(c_ref[...], wht_ref[...], dn,
                              preferred_element_type=jnp.float32)) * 0.5


def _pass1_body(a_ref, c_ref, wt_ref, wh_ref, s_ref, m_ref, l_ref):
    j = pl.program_id(0)

    @pl.when(j == 0)
    def _():
        m_ref[...] = jnp.full_like(m_ref, -jnp.inf)
        l_ref[...] = jnp.zeros_like(l_ref)

    s = _block_scores(a_ref, c_ref, wt_ref, wh_ref)
    s_ref[...] = s
    col = lax.broadcasted_iota(jnp.int32, s.shape, 1) + j * _BN
    valid = col < _N
    m_old = m_ref[:, 0:1]
    bmax = jnp.max(jnp.where(valid, s, -jnp.inf), axis=1, keepdims=True)
    m_new = jnp.maximum(m_old, bmax)
    p = jnp.where(valid, jnp.exp(s - m_new), 0.0)
    l_new = l_ref[:, 0:1] * jnp.exp(m_old - m_new) + jnp.sum(p, axis=1,
                                                             keepdims=True)
    m_ref[...] = jnp.broadcast_to(m_new, m_ref.shape)
    l_ref[...] = jnp.broadcast_to(l_new, l_ref.shape)


def _pass2_body(a_ref, c_ref, m_ref, l_ref, wt_ref, wh_ref, att_ref):
    s = _block_scores(a_ref, c_ref, wt_ref, wh_ref)
    m = m_ref[:, 0:1]
    rl = 1.0 / l_ref[:, 0:1]
    att_ref[...] = jnp.exp(s - m) * rl


def _cols_body(b_ref, n_ref):
    g = pl.program_id(0)
    j = pl.program_id(1)
    r = lax.broadcasted_iota(jnp.int32, b_ref.shape, 0)
    lane = lax.broadcasted_iota(jnp.int32, b_ref.shape, 1)
    q = (j * 5000 + r) * 128 + lane
    x = q >> 5
    b_loc = (x * 21475) >> 26
    n_ref[...] = (q - b_loc * _N).astype(jnp.float32)
    b_ref[...] = (b_loc + 32 * g).astype(jnp.float32)


def _nb():
    return (_N + _BN - 1) // _BN


def kernel(src_idx, rel_idx, Wh, Wt, Wf, Wi):
    a, c = _sc_gather_products(src_idx, rel_idx, Wh, Wt, Wf, Wi)
    wt_t, wh_t = Wt.T, Wh.T
    nb = _nb()

    full = lambda shape: pl.BlockSpec(shape, lambda j: (0, 0))
    scores, m, l = pl.pallas_call(
        _pass1_body,
        grid=(nb,),
        in_specs=[
            full((_B, _D)),
            full((_B, _D)),
            pl.BlockSpec((_D, _BN), lambda j: (0, j)),
            pl.BlockSpec((_D, _BN), lambda j: (0, j)),
        ],
        out_specs=[
            pl.BlockSpec((_B, _BN), lambda j: (0, j)),
            full((_B, 128)),
            full((_B, 128)),
        ],
        out_shape=[
            jax.ShapeDtypeStruct((_B, _N), jnp.float32),
            jax.ShapeDtypeStruct((_B, 128), jnp.float32),
            jax.ShapeDtypeStruct((_B, 128), jnp.float32),
        ],
    )(a, c, wt_t, wh_t)

    att = pl.pallas_call(
        _pass2_body,
        grid=(nb,),
        in_specs=[
            full((_B, _D)),
            full((_B, _D)),
            full((_B, 128)),
            full((_B, 128)),
            pl.BlockSpec((_D, _BN), lambda j: (0, j)),
            pl.BlockSpec((_D, _BN), lambda j: (0, j)),
        ],
        out_specs=pl.BlockSpec((_B, _BN), lambda j: (0, j)),
        out_shape=jax.ShapeDtypeStruct((_B, _N), jnp.float32),
    )(a, c, m, l, wt_t, wh_t)

    ncols = _B * _N // 128
    bcol2, ncol2 = pl.pallas_call(
        _cols_body,
        grid=(_B // 32, 5),
        out_specs=[
            pl.BlockSpec((5000, 128), lambda g, j: (g * 5 + j, 0)),
            pl.BlockSpec((5000, 128), lambda g, j: (g * 5 + j, 0)),
        ],
        out_shape=[
            jax.ShapeDtypeStruct((ncols, 128), jnp.float32),
            jax.ShapeDtypeStruct((ncols, 128), jnp.float32),
        ],
    )()
    cur = jnp.stack([bcol2.reshape(-1), ncol2.reshape(-1)], axis=1)

    return scores, att.reshape(-1), cur
